# baseline (speedup 1.0000x reference)
import functools

import jax
import jax.numpy as jnp
from jax import lax
from jax.experimental import pallas as pl
from jax.experimental.pallas import tpu as pltpu
from jax.experimental.pallas import tpu_sc as plsc

_B = 16384
_L = 50
_D = 64
_SC_CORES = 2
_SC_SUBCORES = 16
_NW = _SC_CORES * _SC_SUBCORES
_UPW = _B // _NW
_NBUF = 8
_BK = 1024


def _sc_interest_sum(interests, table):
    mesh = plsc.VectorSubcoreMesh(
        core_axis_name="c", subcore_axis_name="s",
        num_cores=_SC_CORES, num_subcores=_SC_SUBCORES)

    @functools.partial(
        pl.kernel,
        out_type=jax.ShapeDtypeStruct((_B, _D), jnp.float32),
        mesh=mesh,
        scratch_types=[
            pltpu.VMEM((_UPW, _L), jnp.int32),
            pltpu.VMEM((_UPW, _D), jnp.float32),
            pltpu.VMEM((_NBUF, _L, _D), jnp.float32),
        ] + [pltpu.SemaphoreType.DMA] * _NBUF,
        compiler_params=pltpu.CompilerParams(use_tc_tiling_on_sc=False),
    )
    def run(interests_hbm, table_hbm, out_hbm, idx_v, out_v, rows_v, *sems):
        wid = lax.axis_index("s") * _SC_CORES + lax.axis_index("c")
        base = wid * _UPW
        pltpu.sync_copy(interests_hbm.at[pl.ds(base, _UPW)], idx_v)

        def gather(u, b):
            return pltpu.make_async_copy(
                table_hbm.at[idx_v.at[u]], rows_v.at[b], sems[b])

        for b in range(_NBUF):
            gather(b, b).start()

        def body(g, carry):
            for b in range(_NBUF):
                u = g * _NBUF + b
                gather(u, b).wait()
                for k in range(_D // 16):
                    terms = [rows_v[b, j, pl.ds(16 * k, 16)] for j in range(_L)]
                    while len(terms) > 1:
                        nxt = [terms[i] + terms[i + 1]
                               for i in range(0, len(terms) - 1, 2)]
                        if len(terms) % 2:
                            nxt.append(terms[-1])
                        terms = nxt
                    out_v[u, pl.ds(16 * k, 16)] = terms[0]

                @pl.when(u + _NBUF < _UPW)
                def _():
                    gather(u + _NBUF, b).start()
            return carry

        lax.fori_loop(0, _UPW // _NBUF, body, 0)
        pltpu.sync_copy(out_v, out_hbm.at[pl.ds(base, _UPW)])

    return run(interests, table)


def _bn_stats_body(ctr_ref, saves_ref, out_ref):
    c = jnp.log1p(ctr_ref[:])
    s = jnp.log1p(saves_ref[:])
    out_ref[0, 0] = jnp.sum(c)
    out_ref[0, 1] = jnp.sum(c * c)
    out_ref[1, 0] = jnp.sum(s)
    out_ref[1, 1] = jnp.sum(s * s)


def _bn_stats(ctr, saves):
    return pl.pallas_call(
        _bn_stats_body,
        out_shape=jax.ShapeDtypeStruct((2, 2), jnp.float32),
        out_specs=pl.BlockSpec(memory_space=pltpu.SMEM),
    )(ctr.reshape(128, 128), saves.reshape(128, 128))


def _pre_body(stats_ref, gamma_ref, beta_ref, gender_ref, country_ref,
              ctr_ref, saves_ref, gemb_ref, cemb_ref, wg_ref, wc_ref,
              wct_ref, b_ref, out_ref):
    hp = lax.Precision.HIGHEST
    n = jnp.float32(_B)
    m0 = stats_ref[0, 0] / n
    v0 = stats_ref[0, 1] / n - m0 * m0
    m1 = stats_ref[1, 0] / n
    v1 = stats_ref[1, 1] / n - m1 * m1
    sc0 = lax.rsqrt(v0 + 1e-5) * gamma_ref[0, 0]
    sc1 = lax.rsqrt(v1 + 1e-5) * gamma_ref[0, 1]
    b0 = beta_ref[0, 0] - m0 * sc0
    b1 = beta_ref[0, 1] - m1 * sc1
    wp = jnp.concatenate([wct_ref[0:1, :] * sc0, wct_ref[1:2, :] * sc1], 0)
    bias = b_ref[:] + b0 * wct_ref[0:1, :] + b1 * wct_ref[1:2, :]
    z2 = jnp.concatenate(
        [jnp.log1p(ctr_ref[:]), jnp.log1p(saves_ref[:])], axis=1)

    goh = (gender_ref[:] == lax.broadcasted_iota(
        jnp.int32, (_BK, 8), 1)).astype(jnp.float32)
    coh = (country_ref[:] == lax.broadcasted_iota(
        jnp.int32, (_BK, 256), 1)).astype(jnp.float32)
    ag = lax.dot_general(gemb_ref[:], wg_ref[:],
                         (((1,), (1,)), ((), ())), precision=hp)
    ac = lax.dot_general(cemb_ref[:], wc_ref[:],
                         (((1,), (1,)), ((), ())), precision=hp)
    x = lax.dot_general(goh, ag, (((1,), (0,)), ((), ())), precision=hp)
    x = x + lax.dot_general(coh, ac, (((1,), (0,)), ((), ())), precision=hp)
    x = x + lax.dot_general(z2, wp, (((1,), (0,)), ((), ())), precision=hp)
    out_ref[:] = x + bias


def _dense_pre(stats, gamma, beta, gender, country, ctr, saves,
               gemb_p, cemb_p, wg, wc, wct, fc_b):
    row = lambda i: (i, 0)
    rep = lambda i: (0, 0)
    return pl.pallas_call(
        _pre_body,
        grid=(_B // _BK,),
        in_specs=[
            pl.BlockSpec(memory_space=pltpu.SMEM),
            pl.BlockSpec(memory_space=pltpu.SMEM),
            pl.BlockSpec(memory_space=pltpu.SMEM),
            pl.BlockSpec((_BK, 1), row),
            pl.BlockSpec((_BK, 1), row),
            pl.BlockSpec((_BK, 1), row),
            pl.BlockSpec((_BK, 1), row),
            pl.BlockSpec((8, _D), rep),
            pl.BlockSpec((256, _D), rep),
            pl.BlockSpec((_D, _D), rep),
            pl.BlockSpec((_D, _D), rep),
            pl.BlockSpec((2, _D), rep),
            pl.BlockSpec((1, _D), rep),
        ],
        out_specs=pl.BlockSpec((_BK, _D), row),
        out_shape=jax.ShapeDtypeStruct((_B, _D), jnp.float32),
    )(stats, gamma, beta, gender, country, ctr, saves,
      gemb_p, cemb_p, wg, wc, wct, fc_b)


def _post_body(xpre_ref, isum_ref, vl_ref, wi_ref, out_ref):
    hp = lax.Precision.HIGHEST
    isc = isum_ref[:] / vl_ref[:]
    x = xpre_ref[:] + lax.dot_general(
        isc, wi_ref[:], (((1,), (1,)), ((), ())), precision=hp)
    x = jnp.maximum(x, 0.0)
    s = jnp.sum(x * x, axis=1, keepdims=True)
    out_ref[:] = x / jnp.maximum(jnp.sqrt(s), 1e-12)


def _dense_post(xpre, isum, vl, wi):
    row = lambda i: (i, 0)
    rep = lambda i: (0, 0)
    return pl.pallas_call(
        _post_body,
        grid=(_B // _BK,),
        in_specs=[
            pl.BlockSpec((_BK, _D), row),
            pl.BlockSpec((_BK, _D), row),
            pl.BlockSpec((_BK, 1), row),
            pl.BlockSpec((_D, _D), rep),
        ],
        out_specs=pl.BlockSpec((_BK, _D), row),
        out_shape=jax.ShapeDtypeStruct((_B, _D), jnp.float32),
    )(xpre, isum, vl, wi)


def kernel(gender, country, interests, ctr, saves, valid_lens, gender_embed,
           country_embed, interest_embed, bn_gamma, bn_beta, fc_W, fc_b):
    gender = gender.astype(jnp.int32).reshape(_B, 1)
    country = country.astype(jnp.int32).reshape(_B, 1)
    interests = interests.astype(jnp.int32)

    isum = _sc_interest_sum(interests, interest_embed)
    stats = _bn_stats(ctr, saves)

    gemb_p = jnp.zeros((8, _D), jnp.float32).at[:4, :].set(gender_embed)
    cemb_p = jnp.zeros((256, _D), jnp.float32).at[:200, :].set(country_embed)
    wg = fc_W[:, 0:64]
    wc = fc_W[:, 64:128]
    wi = fc_W[:, 128:192]
    wct = fc_W[:, 192:194].T

    xpre = _dense_pre(stats, bn_gamma.reshape(1, 2), bn_beta.reshape(1, 2),
                      gender, country, ctr, saves, gemb_p, cemb_p, wg, wc,
                      wct, fc_b.reshape(1, _D))
    return _dense_post(xpre, isum, valid_lens.reshape(_B, 1), wi)


# speedup vs baseline: 1.2229x; 1.2229x over previous
import functools

import jax
import jax.numpy as jnp
from jax import lax
from jax.experimental import pallas as pl
from jax.experimental.pallas import tpu as pltpu
from jax.experimental.pallas import tpu_sc as plsc

_B = 16384
_L = 50
_D = 64
_SC_CORES = 2
_SC_SUBCORES = 16
_NW = _SC_CORES * _SC_SUBCORES
_UPW = _B // _NW
_NBUF = 4
_BK = 2048


def _sc_interest_sum(interests, table):
    mesh = plsc.VectorSubcoreMesh(
        core_axis_name="c", subcore_axis_name="s",
        num_cores=_SC_CORES, num_subcores=_SC_SUBCORES)

    @functools.partial(
        pl.kernel,
        out_type=jax.ShapeDtypeStruct((_B, _D), jnp.float32),
        mesh=mesh,
        scratch_types=[
            pltpu.VMEM((_UPW, _L), jnp.int32),
            pltpu.VMEM((_UPW, _D), jnp.float32),
            pltpu.VMEM((_NBUF, _L, _D), jnp.float32),
        ] + [pltpu.SemaphoreType.DMA] * _NBUF,
        compiler_params=pltpu.CompilerParams(use_tc_tiling_on_sc=False),
    )
    def run(interests_hbm, table_hbm, out_hbm, idx_v, out_v, rows_v, *sems):
        wid = lax.axis_index("s") * _SC_CORES + lax.axis_index("c")
        base = wid * _UPW
        pltpu.sync_copy(interests_hbm.at[pl.ds(base, _UPW)], idx_v)

        def gather(u, b):
            return pltpu.make_async_copy(
                table_hbm.at[idx_v.at[u]], rows_v.at[b], sems[b])

        for b in range(_NBUF):
            gather(b, b).start()

        def body(g, carry):
            for b in range(_NBUF):
                u = g * _NBUF + b
                gather(u, b).wait()
                for k in range(_D // 16):
                    terms = [rows_v[b, j, pl.ds(16 * k, 16)] for j in range(_L)]
                    while len(terms) > 1:
                        nxt = [terms[i] + terms[i + 1]
                               for i in range(0, len(terms) - 1, 2)]
                        if len(terms) % 2:
                            nxt.append(terms[-1])
                        terms = nxt
                    out_v[u, pl.ds(16 * k, 16)] = terms[0]

                @pl.when(u + _NBUF < _UPW)
                def _():
                    gather(u + _NBUF, b).start()
            return carry

        lax.fori_loop(0, _UPW // _NBUF, body, 0)
        pltpu.sync_copy(out_v, out_hbm.at[pl.ds(base, _UPW)])

    return run(interests, table)


def _bn_stats_body(ctr_ref, saves_ref, out_ref):
    c = jnp.log1p(ctr_ref[:])
    s = jnp.log1p(saves_ref[:])
    out_ref[0, 0] = jnp.sum(c)
    out_ref[0, 1] = jnp.sum(c * c)
    out_ref[1, 0] = jnp.sum(s)
    out_ref[1, 1] = jnp.sum(s * s)


def _bn_stats(ctr, saves):
    return pl.pallas_call(
        _bn_stats_body,
        out_shape=jax.ShapeDtypeStruct((2, 2), jnp.float32),
        out_specs=pl.BlockSpec(memory_space=pltpu.SMEM),
    )(ctr.reshape(128, 128), saves.reshape(128, 128))


def _pre_body(stats_ref, gamma_ref, beta_ref, gender_ref, country_ref,
              ctr_ref, saves_ref, gemb_ref, cemb_ref, wg_ref, wc_ref,
              wct_ref, b_ref, out_ref):
    hp = lax.Precision.HIGHEST
    n = jnp.float32(_B)
    m0 = stats_ref[0, 0] / n
    v0 = stats_ref[0, 1] / n - m0 * m0
    m1 = stats_ref[1, 0] / n
    v1 = stats_ref[1, 1] / n - m1 * m1
    sc0 = lax.rsqrt(v0 + 1e-5) * gamma_ref[0, 0]
    sc1 = lax.rsqrt(v1 + 1e-5) * gamma_ref[0, 1]
    b0 = beta_ref[0, 0] - m0 * sc0
    b1 = beta_ref[0, 1] - m1 * sc1
    wp = jnp.concatenate([wct_ref[0:1, :] * sc0, wct_ref[1:2, :] * sc1], 0)
    bias = b_ref[:] + b0 * wct_ref[0:1, :] + b1 * wct_ref[1:2, :]
    z2 = jnp.concatenate(
        [jnp.log1p(ctr_ref[:]), jnp.log1p(saves_ref[:])], axis=1)

    goh = (gender_ref[:] == lax.broadcasted_iota(
        jnp.int32, (_BK, 8), 1)).astype(jnp.float32)
    coh = (country_ref[:] == lax.broadcasted_iota(
        jnp.int32, (_BK, 256), 1)).astype(jnp.float32)
    ag = lax.dot_general(gemb_ref[:], wg_ref[:],
                         (((1,), (1,)), ((), ())), precision=hp)
    ac = lax.dot_general(cemb_ref[:], wc_ref[:],
                         (((1,), (1,)), ((), ())), precision=hp)
    x = lax.dot_general(goh, ag, (((1,), (0,)), ((), ())), precision=hp)
    x = x + lax.dot_general(coh, ac, (((1,), (0,)), ((), ())), precision=hp)
    x = x + lax.dot_general(z2, wp, (((1,), (0,)), ((), ())), precision=hp)
    out_ref[:] = x + bias


def _dense_pre(stats, gamma, beta, gender, country, ctr, saves,
               gemb_p, cemb_p, wg, wc, wct, fc_b):
    row = lambda i: (i, 0)
    rep = lambda i: (0, 0)
    return pl.pallas_call(
        _pre_body,
        grid=(_B // _BK,),
        in_specs=[
            pl.BlockSpec(memory_space=pltpu.SMEM),
            pl.BlockSpec(memory_space=pltpu.SMEM),
            pl.BlockSpec(memory_space=pltpu.SMEM),
            pl.BlockSpec((_BK, 1), row),
            pl.BlockSpec((_BK, 1), row),
            pl.BlockSpec((_BK, 1), row),
            pl.BlockSpec((_BK, 1), row),
            pl.BlockSpec((8, _D), rep),
            pl.BlockSpec((256, _D), rep),
            pl.BlockSpec((_D, _D), rep),
            pl.BlockSpec((_D, _D), rep),
            pl.BlockSpec((2, _D), rep),
            pl.BlockSpec((1, _D), rep),
        ],
        out_specs=pl.BlockSpec((_BK, _D), row),
        out_shape=jax.ShapeDtypeStruct((_B, _D), jnp.float32),
    )(stats, gamma, beta, gender, country, ctr, saves,
      gemb_p, cemb_p, wg, wc, wct, fc_b)


def _post_body(xpre_ref, isum_ref, vl_ref, wi_ref, out_ref):
    hp = lax.Precision.HIGHEST
    isc = isum_ref[:] / vl_ref[:]
    x = xpre_ref[:] + lax.dot_general(
        isc, wi_ref[:], (((1,), (1,)), ((), ())), precision=hp)
    x = jnp.maximum(x, 0.0)
    s = jnp.sum(x * x, axis=1, keepdims=True)
    out_ref[:] = x / jnp.maximum(jnp.sqrt(s), 1e-12)


def _dense_post(xpre, isum, vl, wi):
    row = lambda i: (i, 0)
    rep = lambda i: (0, 0)
    return pl.pallas_call(
        _post_body,
        grid=(_B // _BK,),
        in_specs=[
            pl.BlockSpec((_BK, _D), row),
            pl.BlockSpec((_BK, _D), row),
            pl.BlockSpec((_BK, 1), row),
            pl.BlockSpec((_D, _D), rep),
        ],
        out_specs=pl.BlockSpec((_BK, _D), row),
        out_shape=jax.ShapeDtypeStruct((_B, _D), jnp.float32),
    )(xpre, isum, vl, wi)


def kernel(gender, country, interests, ctr, saves, valid_lens, gender_embed,
           country_embed, interest_embed, bn_gamma, bn_beta, fc_W, fc_b):
    gender = gender.astype(jnp.int32).reshape(_B, 1)
    country = country.astype(jnp.int32).reshape(_B, 1)
    interests = interests.astype(jnp.int32)

    isum = _sc_interest_sum(interests, interest_embed)
    stats = _bn_stats(ctr, saves)

    gemb_p = jnp.zeros((8, _D), jnp.float32).at[:4, :].set(gender_embed)
    cemb_p = jnp.zeros((256, _D), jnp.float32).at[:200, :].set(country_embed)
    wg = fc_W[:, 0:64]
    wc = fc_W[:, 64:128]
    wi = fc_W[:, 128:192]
    wct = fc_W[:, 192:194].T

    xpre = _dense_pre(stats, bn_gamma.reshape(1, 2), bn_beta.reshape(1, 2),
                      gender, country, ctr, saves, gemb_p, cemb_p, wg, wc,
                      wct, fc_b.reshape(1, _D))
    return _dense_post(xpre, isum, valid_lens.reshape(_B, 1), wi)
